# baseline (speedup 1.0000x reference)
import functools

import numpy as np
import jax
import jax.numpy as jnp
from jax import lax
from jax.experimental import pallas as pl
from jax.experimental.pallas import tpu as pltpu
from jax.experimental.pallas import tpu_sc as plsc

_D = 16
_LANES = 128
_GRP = 4
_NBQ = 5


def _pos_encoding(seq_len, embed_dim):
    pos = np.arange(seq_len)[:, np.newaxis]
    i = np.arange(embed_dim)[np.newaxis, :]
    angle_rates = 1.0 / np.power(10000, 2 * (i // 2) / np.float32(embed_dim))
    a = pos * angle_rates
    a[:, 0::2] = np.sin(a[:, 0::2])
    a[:, 1::2] = np.cos(a[:, 1::2])
    return a.astype(np.float32)


@functools.lru_cache(maxsize=None)
def _build(seq_len, batch, vocab):
    nc, ns = 2, 16
    nw = nc * ns
    assert batch == nw * _LANES
    n_dg = _D // 8
    tile = 8 * _LANES
    ngrp = seq_len // _GRP
    assert seq_len % _GRP == 0 and ngrp % _NBQ == 0 and ngrp >= 2 * _NBQ
    grows = _GRP * _LANES
    gout = _GRP * n_dg * tile

    mesh = plsc.VectorSubcoreMesh(core_axis_name="c", subcore_axis_name="s")

    @functools.partial(
        pl.kernel,
        out_type=jax.ShapeDtypeStruct((seq_len, n_dg, nw, tile), jnp.float32),
        mesh=mesh,
        scratch_types=[
            pltpu.VMEM((seq_len * _LANES,), jnp.int32),
            pltpu.VMEM((_NBQ, grows, _D), jnp.float32),
            pltpu.VMEM((_NBQ, gout), jnp.float32),
            pltpu.VMEM((seq_len, _D), jnp.float32),
            [pltpu.SemaphoreType.DMA] * _NBQ,
            [pltpu.SemaphoreType.DMA] * _NBQ,
        ],
        compiler_params=pltpu.CompilerParams(
            use_tc_tiling_on_sc=False, needs_layout_passes=False),
    )
    def run(xw_hbm, pos_hbm, table_hbm, out_hbm, xblk, gbuf, obuf, pos_v,
            gsems, osems):
        w = lax.axis_index("s") * nc + lax.axis_index("c")
        pltpu.sync_copy(pos_hbm, pos_v)
        pltpu.sync_copy(xw_hbm.at[w], xblk)

        def gather_copy(g, t):
            return pltpu.async_copy(
                table_hbm.at[xblk.at[pl.ds(g * grows, grows)]], gbuf.at[t],
                gsems[t])

        def gather_wait(g, t):
            pltpu.make_async_copy(
                table_hbm.at[xblk.at[pl.ds(g * grows, grows)]], gbuf.at[t],
                gsems[t]).wait()

        def out_copy(g, t, k, dg):
            return pltpu.async_copy(
                obuf.at[t].at[pl.ds((k * n_dg + dg) * tile, tile)],
                out_hbm.at[g * _GRP + k, dg, w], osems[t])

        def out_wait(g, t, k, dg):
            pltpu.make_async_copy(
                obuf.at[t].at[pl.ds((k * n_dg + dg) * tile, tile)],
                out_hbm.at[g * _GRP + k, dg, w], osems[t]).wait()

        iota16 = lax.iota(jnp.int32, 16)
        sidx = (iota16 // 8) * tile + (iota16 % 8) * _LANES

        def compute(g, t):
            gb = gbuf.at[t]
            ofl = obuf.at[t]
            for k in range(_GRP):
                pv = pos_v[g * _GRP + k]
                base = k * n_dg * tile
                roff = k * _LANES

                @plsc.parallel_loop(0, _LANES, unroll=8)
                def vloop(i):
                    vals = gb[roff + i] + pv
                    plsc.store_scatter(ofl, [sidx + (base + i)], vals)

        for t in range(_NBQ - 1):
            gather_copy(t, t)

        def chunk_body(c, carry):
            g_base = c * _NBQ
            for e in range(_NBQ):
                g = g_base + e
                gather_wait(g, e)
                @pl.when(g + _NBQ - 1 < ngrp)
                def _():
                    gather_copy(g + _NBQ - 1, (e - 1) % _NBQ)
                @pl.when(g >= _NBQ)
                def _():
                    for k in range(_GRP):
                        for dg in range(n_dg):
                            out_wait(g - _NBQ, e, k, dg)
                compute(g, e)
                for k in range(_GRP):
                    for dg in range(n_dg):
                        out_copy(g, e, k, dg)
            return carry

        lax.fori_loop(0, ngrp // _NBQ, chunk_body, 0)
        for e in range(_NBQ):
            for k in range(_GRP):
                for dg in range(n_dg):
                    out_wait(ngrp - _NBQ + e, e, k, dg)

    return run


def kernel(x_in, table):
    b, s = x_in.shape
    vocab, d = table.shape
    nw = b // _LANES
    xw = (x_in.T.astype(jnp.int32)
          .reshape(s, nw, _LANES).transpose((1, 0, 2)).reshape(nw, s * _LANES))
    pos = jnp.asarray(_pos_encoding(s, d))
    out_lin = _build(s, b, vocab)(xw, pos, table)
    out5 = out_lin.reshape(s, d // 8, nw, 8, _LANES)
    return out5.transpose((2, 4, 0, 1, 3)).reshape(nw * _LANES, s, d)


# speedup vs baseline: 2.2467x; 2.2467x over previous
import functools

import numpy as np
import jax
import jax.numpy as jnp
from jax import lax
from jax.experimental import pallas as pl
from jax.experimental.pallas import tpu as pltpu
from jax.experimental.pallas import tpu_sc as plsc

_D = 16
_LANES = 128
_HB = 2048
_NIQ = 4


def _pos_encoding(seq_len, embed_dim):
    pos = np.arange(seq_len)[:, np.newaxis]
    i = np.arange(embed_dim)[np.newaxis, :]
    angle_rates = 1.0 / np.power(10000, 2 * (i // 2) / np.float32(embed_dim))
    a = pos * angle_rates
    a[:, 0::2] = np.sin(a[:, 0::2])
    a[:, 1::2] = np.cos(a[:, 1::2])
    return a.astype(np.float32)


@functools.lru_cache(maxsize=None)
def _build(seq_len, batch, vocab):
    nc, ns = 2, 16
    assert ns == _D and batch % _HB == 0 and seq_len % nc == 0
    nct = batch // _LANES
    hct = _HB // _LANES
    nh = batch // _HB
    s_half = seq_len // nc
    nsteps = s_half * nh
    assert nsteps % _NIQ == 0

    mesh = plsc.VectorSubcoreMesh(core_axis_name="c", subcore_axis_name="s")

    @functools.partial(
        pl.kernel,
        out_type=jax.ShapeDtypeStruct((seq_len, _D // 8, nct, 8, _LANES),
                                      jnp.float32),
        mesh=mesh,
        scratch_types=[
            pltpu.VMEM((vocab,), jnp.float32),
            pltpu.VMEM((_NIQ, _HB), jnp.int32),
            pltpu.VMEM((2, hct, _LANES), jnp.float32),
            pltpu.VMEM((seq_len, _D), jnp.float32),
            [pltpu.SemaphoreType.DMA] * _NIQ,
            [pltpu.SemaphoreType.DMA] * 2,
        ],
        compiler_params=pltpu.CompilerParams(
            use_tc_tiling_on_sc=False, needs_layout_passes=False),
    )
    def run(x_hbm, psp_hbm, tabt_hbm, out_hbm, row_v, xbuf, obuf, psp_v,
            isems, osems):
        d = lax.axis_index("s")
        half = lax.axis_index("c")
        s0 = half * s_half
        dg = d // 8
        r = d % 8
        pltpu.sync_copy(psp_hbm.at[d], psp_v)
        pltpu.sync_copy(tabt_hbm.at[d], row_v)

        def h_to_sb(h):
            return s0 + h // nh, (h % nh) * _HB

        def idx_copy(h, t):
            s, b0 = h_to_sb(h)
            return pltpu.async_copy(
                x_hbm.at[s].at[pl.ds(b0, _HB)], xbuf.at[t], isems[t])

        def idx_wait(h, t):
            s, b0 = h_to_sb(h)
            pltpu.make_async_copy(
                x_hbm.at[s].at[pl.ds(b0, _HB)], xbuf.at[t], isems[t]).wait()

        def out_dst(h):
            s, b0 = h_to_sb(h)
            return out_hbm.at[s, dg, pl.ds((b0 // _LANES), hct), r]

        def out_copy(h, ot):
            return pltpu.async_copy(obuf.at[ot], out_dst(h), osems[ot])

        def out_wait(h, ot):
            pltpu.make_async_copy(obuf.at[ot], out_dst(h), osems[ot]).wait()

        def compute(h, t, ot):
            s, _ = h_to_sb(h)
            pv = psp_v[s]
            xb = xbuf.at[t]
            ob = obuf.at[ot]

            @plsc.parallel_loop(0, _HB // 16, unroll=8)
            def vloop(g):
                idx16 = xb[pl.ds(g * 16, 16)]
                vals = plsc.load_gather(row_v, [idx16])
                ob[g // 8, pl.ds((g % 8) * 16, 16)] = vals + pv

        for t in range(_NIQ):
            idx_copy(t, t)

        def chunk_body(c4, carry):
            h_base = c4 * _NIQ
            for e in range(_NIQ):
                h = h_base + e
                ot = e % 2
                idx_wait(h, e)
                @pl.when(h >= 2)
                def _():
                    out_wait(h - 2, ot)
                compute(h, e, ot)
                out_copy(h, ot)
                @pl.when(h + _NIQ < nsteps)
                def _():
                    idx_copy(h + _NIQ, e)
            return carry

        lax.fori_loop(0, nsteps // _NIQ, chunk_body, 0)
        for h, ot in ((nsteps - 2, 0), (nsteps - 1, 1)):
            out_wait(h, ot)

    return run


def kernel(x_in, table):
    b, s = x_in.shape
    vocab, d = table.shape
    x_t = x_in.T.astype(jnp.int32)
    table_t = table.T
    pos = _pos_encoding(s, d)
    psp = jnp.asarray(np.tile(pos.T[:, :, None], (1, 1, 16)))
    out_lin = _build(s, b, vocab)(x_t, psp, table_t)
    nw = out_lin.shape[2]
    return out_lin.transpose((2, 4, 0, 1, 3)).reshape(nw * _LANES, s, d)


# speedup vs baseline: 2.3673x; 1.0537x over previous
import functools

import numpy as np
import jax
import jax.numpy as jnp
from jax import lax
from jax.experimental import pallas as pl
from jax.experimental.pallas import tpu as pltpu
from jax.experimental.pallas import tpu_sc as plsc

_D = 16
_LANES = 128
_HB = 2048
_NIQ = 4


def _pos_encoding(seq_len, embed_dim):
    pos = np.arange(seq_len)[:, np.newaxis]
    i = np.arange(embed_dim)[np.newaxis, :]
    angle_rates = 1.0 / np.power(10000, 2 * (i // 2) / np.float32(embed_dim))
    a = pos * angle_rates
    a[:, 0::2] = np.sin(a[:, 0::2])
    a[:, 1::2] = np.cos(a[:, 1::2])
    return a.astype(np.float32)


@functools.lru_cache(maxsize=None)
def _build(seq_len, batch, vocab):
    nc, ns = 2, 16
    assert ns == _D and batch % _HB == 0 and seq_len % nc == 0
    nct = batch // _LANES
    hct = _HB // _LANES
    nh = batch // _HB
    s_half = seq_len // nc
    nsteps = s_half * nh
    assert nsteps % _NIQ == 0

    mesh = plsc.VectorSubcoreMesh(core_axis_name="c", subcore_axis_name="s")

    @functools.partial(
        pl.kernel,
        out_type=jax.ShapeDtypeStruct((seq_len, _D // 8, nct, 8, _LANES),
                                      jnp.float32),
        mesh=mesh,
        scratch_types=[
            pltpu.VMEM((vocab,), jnp.float32),
            pltpu.VMEM((_NIQ, _HB // _LANES, _LANES), jnp.int32),
            pltpu.VMEM((2, hct, _LANES), jnp.float32),
            pltpu.VMEM((seq_len, _D), jnp.float32),
            [pltpu.SemaphoreType.DMA] * _NIQ,
            [pltpu.SemaphoreType.DMA] * 2,
        ],
        compiler_params=pltpu.CompilerParams(
            use_tc_tiling_on_sc=False, needs_layout_passes=False),
    )
    def run(x_hbm, psp_hbm, tabt_hbm, out_hbm, row_v, xbuf, obuf, psp_v,
            isems, osems):
        d = lax.axis_index("s")
        half = lax.axis_index("c")
        s0 = half * s_half
        dg = d // 8
        r = d % 8
        pltpu.sync_copy(psp_hbm.at[d], psp_v)
        pltpu.sync_copy(tabt_hbm.at[d], row_v)

        def h_to_sb(h):
            return s0 + h // nh, (h % nh) * _HB

        def idx_src(h):
            s, b0 = h_to_sb(h)
            return x_hbm.at[s // 8, pl.ds(b0 // _LANES, hct), s % 8]

        def idx_copy(h, t):
            return pltpu.async_copy(idx_src(h), xbuf.at[t], isems[t])

        def idx_wait(h, t):
            pltpu.make_async_copy(idx_src(h), xbuf.at[t], isems[t]).wait()

        def out_dst(h):
            s, b0 = h_to_sb(h)
            return out_hbm.at[s, dg, pl.ds((b0 // _LANES), hct), r]

        def out_copy(h, ot):
            return pltpu.async_copy(obuf.at[ot], out_dst(h), osems[ot])

        def out_wait(h, ot):
            pltpu.make_async_copy(obuf.at[ot], out_dst(h), osems[ot]).wait()

        def compute(h, t, ot):
            s, _ = h_to_sb(h)
            pv = psp_v[s]
            xb = xbuf.at[t]
            ob = obuf.at[ot]

            @plsc.parallel_loop(0, _HB // 16, unroll=8)
            def vloop(g):
                idx16 = xb[g // 8, pl.ds((g % 8) * 16, 16)]
                vals = plsc.load_gather(row_v, [idx16])
                ob[g // 8, pl.ds((g % 8) * 16, 16)] = vals + pv

        for t in range(_NIQ):
            idx_copy(t, t)

        def chunk_body(c4, carry):
            h_base = c4 * _NIQ
            for e in range(_NIQ):
                h = h_base + e
                ot = e % 2
                idx_wait(h, e)
                @pl.when(h >= 2)
                def _():
                    out_wait(h - 2, ot)
                compute(h, e, ot)
                out_copy(h, ot)
                @pl.when(h + _NIQ < nsteps)
                def _():
                    idx_copy(h + _NIQ, e)
            return carry

        lax.fori_loop(0, nsteps // _NIQ, chunk_body, 0)
        for h, ot in ((nsteps - 2, 0), (nsteps - 1, 1)):
            out_wait(h, ot)

    return run


def kernel(x_in, table):
    b, s = x_in.shape
    vocab, d = table.shape
    x4 = (x_in.T.astype(jnp.int32)
          .reshape(s // 8, 8, b // _LANES, _LANES).transpose((0, 2, 1, 3)))
    table_t = table.T
    pos = _pos_encoding(s, d)
    psp = jnp.asarray(np.tile(pos.T[:, :, None], (1, 1, 16)))
    out_lin = _build(s, b, vocab)(x4, psp, table_t)
    nw = out_lin.shape[2]
    return out_lin.transpose((2, 4, 0, 1, 3)).reshape(nw * _LANES, s, d)
